# Initial kernel scaffold; baseline (speedup 1.0000x reference)
import jax
import jax.numpy as jnp
from jax.experimental import pallas as pl


def kernel(x, feature_grids, grid_scales, grid_translations):
    raise NotImplementedError("write your pallas kernel here")


# speedup vs baseline: 2.8913x; 2.8913x over previous
import functools

import jax
import jax.numpy as jnp
from jax import lax
from jax.experimental import pallas as pl
from jax.experimental.pallas import tpu as pltpu
from jax.experimental.pallas import tpu_sc as plsc

G = 64
C = 2
E = 64
B_REAL = 100000
NWORKERS = 32
PW = 3136
B_PAD = NWORKERS * PW
CH = 448
NCHUNK = PW // CH
NGROUP = CH // 16
GRID_CELLS = E * E * E
SENTINEL = -1

_mesh = plsc.VectorSubcoreMesh(core_axis_name="c", subcore_axis_name="s")


def _floor_parts(v):
    i = v.astype(jnp.int32)
    fi = i.astype(jnp.float32)
    f0 = jnp.where(fi > v, fi - 1.0, fi)
    return f0, v - f0


def _axis_terms(coord):
    f0, frac = _floor_parts(coord)
    lo_ok = (f0 >= 0.0) & (f0 <= 63.0)
    hi_ok = (f0 >= -1.0) & (f0 <= 62.0)
    li = jnp.clip(f0, 0.0, 63.0).astype(jnp.int32)
    hi = jnp.clip(f0 + 1.0, 0.0, 63.0).astype(jnp.int32)
    wlo = jnp.where(lo_ok, 1.0 - frac, 0.0)
    whi = jnp.where(hi_ok, frac, 0.0)
    return li, hi, wlo, whi


@functools.partial(
    pl.kernel,
    mesh=_mesh,
    compiler_params=pltpu.CompilerParams(needs_layout_passes=False),
    out_type=jax.ShapeDtypeStruct((B_PAD, G * C), jnp.float32),
    scratch_types=(
        [pltpu.VMEM((PW,), jnp.float32) for _ in range(3)]
        + [pltpu.VMEM((6, G, 16), jnp.float32)]
        + [pltpu.VMEM((CH,), jnp.int32) for _ in range(8)]
        + [pltpu.VMEM((8, CH), jnp.float32)]
        + [pltpu.VMEM((CH,), jnp.int32) for _ in range(8)]
        + [pltpu.VMEM((CH, G * C), jnp.float32)]
        + [pltpu.SemaphoreType.DMA]
    ),
)
def _amg_sc_kernel(table, xt, consts, out, *refs):
    xs_refs = refs[0:3]
    cv = refs[3]
    idx_refs = refs[4:12]
    w_v = refs[12]
    land_refs = refs[13:21]
    acc_v = refs[21]
    sem = refs[22]

    wid = lax.axis_index("s") * 2 + lax.axis_index("c")
    base = wid * PW
    for a in range(3):
        pltpu.sync_copy(xt.at[pl.ds(a * B_PAD + base, PW)], xs_refs[a])
    pltpu.sync_copy(consts, cv)

    lane = lax.iota(jnp.int32, 16)

    def zinit(gr, c0):
        z16 = jnp.zeros((16,), jnp.int32)
        for k in range(8):
            land_refs[k][pl.ds(gr * 16, 16)] = z16
        return c0

    lax.fori_loop(0, NGROUP, zinit, 0)

    def chunk_body(ci, carry):
        cbase = ci * CH

        def grid_body(g, carry2):
            ax = cv[0, g]
            bx = cv[1, g]
            ay = cv[2, g]
            by = cv[3, g]
            az = cv[4, g]
            bz = cv[5, g]

            def p1(gr, c3):
                o = cbase + gr * 16
                px = xs_refs[0][pl.ds(o, 16)]
                py = xs_refs[1][pl.ds(o, 16)]
                pz = xs_refs[2][pl.ds(o, 16)]
                ix = px * ax + bx
                iy = py * ay + by
                iz = pz * az + bz
                xli, xhi, wx0, wx1 = _axis_terms(ix)
                yli, yhi, wy0, wy1 = _axis_terms(iy)
                zli, zhi, wz0, wz1 = _axis_terms(iz)
                s = gr * 16
                for dz in range(2):
                    zi = zli if dz == 0 else zhi
                    wz = wz0 if dz == 0 else wz1
                    zrow = zi * (E * E)
                    for dy in range(2):
                        yi = yli if dy == 0 else yhi
                        wy = wy0 if dy == 0 else wy1
                        zyrow = zrow + yi * E
                        wzy = wz * wy
                        for dx in range(2):
                            xi = xli if dx == 0 else xhi
                            wx = wx0 if dx == 0 else wx1
                            k = dz * 4 + dy * 2 + dx
                            w = wzy * wx
                            row = zyrow + xi
                            row = jnp.where(w > 0.0, row, SENTINEL)
                            idx_refs[k][pl.ds(s, 16)] = row
                            w_v[k, pl.ds(s, 16)] = w
                return c3

            lax.fori_loop(0, NGROUP, p1, 0)

            gtab = table.at[pl.ds(g * GRID_CELLS, GRID_CELLS)]
            copies = [
                pltpu.async_copy(
                    gtab.at[plsc.Indices(idx_refs[k], ignored_value=SENTINEL)],
                    land_refs[k],
                    sem,
                )
                for k in range(8)
            ]
            for cp in copies:
                cp.wait()

            def p2(gr, c3):
                o = gr * 16
                pidx = lane + o
                acc0 = jnp.zeros((16,), jnp.float32)
                acc1 = jnp.zeros((16,), jnp.float32)
                for k in range(8):
                    w = w_v[k, pl.ds(o, 16)]
                    word = land_refs[k][pl.ds(o, 16)]
                    both = plsc.bitcast(word, jnp.bfloat16)
                    v0, v1 = plsc.unpack(both, format=plsc.PackFormat.INTERLEAVED)
                    acc0 = acc0 + w * v0
                    acc1 = acc1 + w * v1
                col = jnp.full((16,), 2, jnp.int32) * g
                plsc.store_scatter(acc_v, [pidx, col], acc0)
                plsc.store_scatter(acc_v, [pidx, col + 1], acc1)
                return c3

            lax.fori_loop(0, NGROUP, p2, 0)
            return carry2

        lax.fori_loop(0, G, grid_body, 0)
        pltpu.sync_copy(acc_v, out.at[pl.ds(base + cbase, CH)])
        return carry

    lax.fori_loop(0, NCHUNK, chunk_body, 0)


def kernel(x, feature_grids, grid_scales, grid_translations):
    fg = feature_grids.astype(jnp.bfloat16)
    c0 = lax.bitcast_convert_type(fg[:, 0], jnp.uint16).astype(jnp.uint32)
    c1 = lax.bitcast_convert_type(fg[:, 1], jnp.uint16).astype(jnp.uint32)
    table = lax.bitcast_convert_type(c0 | (c1 << 16), jnp.int32).reshape(-1)
    s = 31.5 / 1.48
    A = (grid_scales * s).astype(jnp.float32)
    Bc = (grid_translations * s + 31.5).astype(jnp.float32)
    consts = jnp.stack(
        [A[:, 0], Bc[:, 0], A[:, 1], Bc[:, 1], A[:, 2], Bc[:, 2]]
    )
    consts = jnp.broadcast_to(consts[:, :, None], (6, G, 16)) + 0.0
    xt = jnp.pad(x, ((0, B_PAD - B_REAL), (0, 0))).T.reshape(-1)
    out = _amg_sc_kernel(table, xt, consts)
    return out[:B_REAL]
